# baseline (speedup 1.0000x reference)
import functools

import jax
import jax.numpy as jnp
from jax import lax
from jax.experimental import pallas as pl
from jax.experimental.pallas import tpu as pltpu
from jax.experimental.pallas import tpu_sc as plsc

NUM_CLASSES = 100000
HIDDEN = 64
BATCH = 16384

_info = plsc.get_sparse_core_info()
NC, NS, L = _info.num_cores, _info.num_subcores, _info.num_lanes
NW = NC * NS
B_PER_W = BATCH // NW
CHUNK = 128
NCHUNK = B_PER_W // CHUNK


def _make_kernel():
  mesh = plsc.VectorSubcoreMesh(core_axis_name="c", subcore_axis_name="s")

  @functools.partial(
      pl.kernel,
      mesh=mesh,
      out_type=jax.ShapeDtypeStruct((BATCH, HIDDEN), jnp.float32),
      compiler_params=pltpu.CompilerParams(use_tc_tiling_on_sc=False),
      scratch_types=[
          pltpu.VMEM((NCHUNK, CHUNK), jnp.int32),
          pltpu.VMEM((B_PER_W, HIDDEN), jnp.float32),
          pltpu.SemaphoreType.DMA,
      ],
  )
  def gather_kernel(idx_hbm, table_hbm, out_hbm, idx_v, rows_v, sem):
    wid = lax.axis_index("s") * NC + lax.axis_index("c")
    base = wid * B_PER_W
    for c in range(NCHUNK):
      pltpu.sync_copy(idx_hbm.at[pl.ds(base + c * CHUNK, CHUNK)], idx_v.at[c])
    copies = []
    for c in range(NCHUNK):
      copies.append(
          pltpu.async_copy(
              table_hbm.at[idx_v.at[c]],
              rows_v.at[pl.ds(c * CHUNK, CHUNK)],
              sem,
          )
      )
    for cp in copies:
      cp.wait()
    pltpu.sync_copy(rows_v, out_hbm.at[pl.ds(base, B_PER_W)])

  return gather_kernel


_gather = _make_kernel()


@jax.jit
def kernel(labels, embedding_table):
  return _gather(jnp.asarray(labels, jnp.int32), embedding_table)


# speedup vs baseline: 2.1327x; 2.1327x over previous
import functools

import jax
import jax.numpy as jnp
from jax import lax
from jax.experimental import pallas as pl
from jax.experimental.pallas import tpu as pltpu
from jax.experimental.pallas import tpu_sc as plsc

NUM_CLASSES = 100000
HIDDEN = 64
BATCH = 16384
VOCAB = NUM_CLASSES + 1

_info = plsc.get_sparse_core_info()
NC, NS, L = _info.num_cores, _info.num_subcores, _info.num_lanes
NW = NC * NS
ROWS_PER_W = HIDDEN // NW
B_HALF = BATCH // 2
UNROLL = 8


def _make_kernel():
  mesh = plsc.VectorSubcoreMesh(core_axis_name="c", subcore_axis_name="s")

  @functools.partial(
      pl.kernel,
      mesh=mesh,
      out_type=jax.ShapeDtypeStruct((HIDDEN, BATCH), jnp.float32),
      compiler_params=pltpu.CompilerParams(needs_layout_passes=False),
      scratch_types=[
          pltpu.VMEM((VOCAB,), jnp.float32),
          pltpu.VMEM((B_HALF,), jnp.int32),
          pltpu.VMEM((B_HALF,), jnp.float32),
          pltpu.SemaphoreType.DMA,
      ],
  )
  def col_gather(lab_hbm, tab_t_hbm, out_t_hbm, row_v, lab_v, out_v, sem):
    wid = lax.axis_index("s") * NC + lax.axis_index("c")
    for r in range(ROWS_PER_W):
      h = wid * ROWS_PER_W + r
      pltpu.async_copy(tab_t_hbm.at[h], row_v, sem).wait()
      for half in range(2):
        base = half * B_HALF
        pltpu.async_copy(lab_hbm.at[pl.ds(base, B_HALF)], lab_v, sem).wait()

        def body(i, _):
          for u in range(UNROLL):
            off = (i * UNROLL + u) * L
            idx = lab_v[pl.ds(off, L)]
            out_v[pl.ds(off, L)] = plsc.load_gather(row_v, [idx])
          return 0

        lax.fori_loop(0, B_HALF // (L * UNROLL), body, 0)
        pltpu.async_copy(out_v, out_t_hbm.at[h, pl.ds(base, B_HALF)], sem).wait()

  return col_gather


_gather = _make_kernel()


@jax.jit
def kernel(labels, embedding_table):
  out_t = _gather(jnp.asarray(labels, jnp.int32), embedding_table.T)
  return out_t.T


# speedup vs baseline: 2.7260x; 1.2782x over previous
import functools

import jax
import jax.numpy as jnp
from jax import lax
from jax.experimental import pallas as pl
from jax.experimental.pallas import tpu as pltpu
from jax.experimental.pallas import tpu_sc as plsc

NUM_CLASSES = 100000
HIDDEN = 64
BATCH = 16384
VOCAB = NUM_CLASSES + 1

_info = plsc.get_sparse_core_info()
NC, NS, L = _info.num_cores, _info.num_subcores, _info.num_lanes
NW = NC * NS
ROWS_PER_W = HIDDEN // NW
BLK = 4096
NBLK = BATCH // BLK


def _make_kernel():
  mesh = plsc.VectorSubcoreMesh(core_axis_name="c", subcore_axis_name="s")

  @functools.partial(
      pl.kernel,
      mesh=mesh,
      out_type=jax.ShapeDtypeStruct((HIDDEN, BATCH), jnp.float32),
      compiler_params=pltpu.CompilerParams(needs_layout_passes=False),
      scratch_types=[
          pltpu.VMEM((VOCAB,), jnp.float32),
          pltpu.VMEM((BATCH,), jnp.int32),
          pltpu.VMEM((2, BLK), jnp.float32),
          pltpu.SemaphoreType.DMA,
          pltpu.SemaphoreType.DMA,
          pltpu.SemaphoreType.DMA,
      ],
  )
  def col_gather(lab_hbm, tab_t_hbm, out_t_hbm, row_v, lab_v, out_v,
                 sem_row, sem_lab, sem_out):
    wid = lax.axis_index("s") * NC + lax.axis_index("c")
    h0 = wid * ROWS_PER_W
    lab_cp = pltpu.async_copy(lab_hbm, lab_v, sem_lab)
    row_cp = pltpu.async_copy(tab_t_hbm.at[h0], row_v, sem_row)
    lab_cp.wait()
    row_cp.wait()
    out_cps = [None, None]
    for r in range(ROWS_PER_W):
      h = h0 + r
      for b in range(NBLK):
        buf = b % 2
        if out_cps[buf] is not None:
          out_cps[buf].wait()

        @plsc.parallel_loop(0, BLK // L, unroll=8)
        def body(i):
          idx = lab_v[pl.ds(b * BLK + i * L, L)]
          out_v[buf, pl.ds(i * L, L)] = plsc.load_gather(row_v, [idx])

        out_cps[buf] = pltpu.async_copy(
            out_v.at[buf], out_t_hbm.at[h, pl.ds(b * BLK, BLK)], sem_out
        )
      if r + 1 < ROWS_PER_W:
        row_cp = pltpu.async_copy(tab_t_hbm.at[h0 + r + 1], row_v, sem_row)
        row_cp.wait()
    for cp in out_cps:
      cp.wait()

  return col_gather


_gather = _make_kernel()


@jax.jit
def kernel(labels, embedding_table):
  out_t = _gather(jnp.asarray(labels, jnp.int32), embedding_table.T)
  return out_t.T
